# Initial kernel scaffold; baseline (speedup 1.0000x reference)
import jax
import jax.numpy as jnp
from jax.experimental import pallas as pl


def kernel(x, gate_W, gate_b, expert_W, expert_b):
    raise NotImplementedError("write your pallas kernel here")


# speedup vs baseline: 1.5650x; 1.5650x over previous
import functools

import jax
import jax.numpy as jnp
from jax.experimental import pallas as pl

HID = 1024
NEXP = 8
TOPK = 2
TOK_TILE = 512
LANES = 128


def _moe_body(xb_ref, xlo_ref, gwt_ref, gwtlo_ref, gb_ref, w_ref, eb_ref,
              out_ref, logits_ref):
    xb = xb_ref[...]
    gwt = gwt_ref[...]
    lg = jnp.dot(xb, gwt, preferred_element_type=jnp.float32)
    lg = lg + jnp.dot(xb, gwtlo_ref[...], preferred_element_type=jnp.float32)
    xlo = xlo_ref[...]
    lg = lg + jnp.dot(xlo, gwt, preferred_element_type=jnp.float32)
    lg = lg + jnp.dot(xlo, gwtlo_ref[...], preferred_element_type=jnp.float32)
    lg = lg + gb_ref[...]
    logits_ref[...] = lg[:, :NEXP]

    lane = jax.lax.broadcasted_iota(jnp.int32, (TOK_TILE, LANES), 1)
    neg = jnp.float32(-1e30)
    lgm = jnp.where(lane < NEXP, lg, neg)
    m0 = jnp.max(lgm, axis=1, keepdims=True)
    e0 = jnp.min(jnp.where(lgm == m0, lane, LANES), axis=1, keepdims=True)
    lg1 = jnp.where(lane == e0, neg, lgm)
    m1 = jnp.max(lg1, axis=1, keepdims=True)
    e1 = jnp.min(jnp.where(lg1 == m1, lane, LANES), axis=1, keepdims=True)
    t = jnp.exp(m1 - m0)
    denom = 1.0 + t
    w0 = 1.0 / denom
    w1 = t / denom

    acc = jnp.zeros((TOK_TILE, HID), jnp.float32)
    for e in range(NEXP):
        we = jnp.where(e0 == e, w0, 0.0) + jnp.where(e1 == e, w1, 0.0)
        y = jax.lax.dot_general(
            xb, w_ref[e], (((1,), (1,)), ((), ())),
            preferred_element_type=jnp.float32)
        y = y + eb_ref[e][None, :]
        acc = acc + we * y
    out_ref[...] = acc


@functools.partial(jax.jit, static_argnums=())
def kernel(x, gate_W, gate_b, expert_W, expert_b):
    bsz, seq, hsz = x.shape
    tokens = bsz * seq
    xf = x.reshape(tokens, hsz)
    xr = xf.astype(jnp.bfloat16)
    xlo = (xf - xr.astype(jnp.float32)).astype(jnp.bfloat16)
    w_bf = expert_W.astype(jnp.bfloat16)
    gwt_f = jnp.zeros((hsz, LANES), jnp.float32).at[:, :NEXP].set(gate_W.T)
    gwt = gwt_f.astype(jnp.bfloat16)
    gwt_lo = (gwt_f - gwt.astype(jnp.float32)).astype(jnp.bfloat16)
    gb = jnp.zeros((1, LANES), jnp.float32).at[0, :NEXP].set(gate_b)

    grid = (tokens // TOK_TILE,)
    out, logits = pl.pallas_call(
        _moe_body,
        grid=grid,
        in_specs=[
            pl.BlockSpec((TOK_TILE, hsz), lambda i: (i, 0)),
            pl.BlockSpec((TOK_TILE, hsz), lambda i: (i, 0)),
            pl.BlockSpec((hsz, LANES), lambda i: (0, 0)),
            pl.BlockSpec((hsz, LANES), lambda i: (0, 0)),
            pl.BlockSpec((1, LANES), lambda i: (0, 0)),
            pl.BlockSpec((NEXP, hsz, hsz), lambda i: (0, 0, 0)),
            pl.BlockSpec((NEXP, hsz), lambda i: (0, 0)),
        ],
        out_specs=[
            pl.BlockSpec((TOK_TILE, hsz), lambda i: (i, 0)),
            pl.BlockSpec((TOK_TILE, NEXP), lambda i: (i, 0)),
        ],
        out_shape=[
            jax.ShapeDtypeStruct((tokens, hsz), jnp.float32),
            jax.ShapeDtypeStruct((tokens, NEXP), jnp.float32),
        ],
    )(xr, xlo, gwt, gwt_lo, gb, w_bf, expert_b)
    return out.reshape(bsz, seq, hsz), logits


# speedup vs baseline: 1.6746x; 1.0700x over previous
import functools

import jax
import jax.numpy as jnp
from jax.experimental import pallas as pl

HID = 1024
NEXP = 8
TOPK = 2
TOK_TILE = 512
LANES = 128


def _moe_body(x_ref, gwt_ref, gwtlo_ref, gb_ref, w_ref, eb_ref,
              out_ref, logits_ref):
    xf = x_ref[...]
    xb = xf.astype(jnp.bfloat16)
    xlo = (xf - xb.astype(jnp.float32)).astype(jnp.bfloat16)

    gwt = gwt_ref[...]
    lg = jnp.dot(xb, gwt, preferred_element_type=jnp.float32)
    lg = lg + jnp.dot(xb, gwtlo_ref[...], preferred_element_type=jnp.float32)
    lg = lg + jnp.dot(xlo, gwt, preferred_element_type=jnp.float32)
    lg = lg + jnp.dot(xlo, gwtlo_ref[...], preferred_element_type=jnp.float32)
    lg = lg + gb_ref[...]
    logits_ref[...] = lg[:, :NEXP]

    lane = jax.lax.broadcasted_iota(jnp.int32, (TOK_TILE, LANES), 1)
    neg = jnp.float32(-1e30)
    lgm = jnp.where(lane < NEXP, lg, neg)
    m0 = jnp.max(lgm, axis=1, keepdims=True)
    e0 = jnp.min(jnp.where(lgm == m0, lane, LANES), axis=1, keepdims=True)
    lg1 = jnp.where(lane == e0, neg, lgm)
    m1 = jnp.max(lg1, axis=1, keepdims=True)
    e1 = jnp.min(jnp.where(lg1 == m1, lane, LANES), axis=1, keepdims=True)
    t = jnp.exp(m1 - m0)
    denom = 1.0 + t
    w0 = 1.0 / denom
    w1 = t / denom

    acc = None
    bias = jnp.zeros((TOK_TILE, HID), jnp.float32)
    for e in range(NEXP):
        we = jnp.where(e0 == e, w0, 0.0) + jnp.where(e1 == e, w1, 0.0)
        xw = xb * we.astype(jnp.bfloat16)
        y = jax.lax.dot_general(
            xw, w_ref[e], (((1,), (1,)), ((), ())),
            preferred_element_type=jnp.float32)
        acc = y if acc is None else acc + y
        bias = bias + we * eb_ref[e][None, :]
    out_ref[...] = acc + bias


@functools.partial(jax.jit, static_argnums=())
def kernel(x, gate_W, gate_b, expert_W, expert_b):
    bsz, seq, hsz = x.shape
    tokens = bsz * seq
    xf = x.reshape(tokens, hsz)
    w_bf = expert_W.astype(jnp.bfloat16)
    gwt_f = jnp.zeros((hsz, LANES), jnp.float32).at[:, :NEXP].set(gate_W.T)
    gwt = gwt_f.astype(jnp.bfloat16)
    gwt_lo = (gwt_f - gwt.astype(jnp.float32)).astype(jnp.bfloat16)
    gb = jnp.zeros((1, LANES), jnp.float32).at[0, :NEXP].set(gate_b)

    grid = (tokens // TOK_TILE,)
    out, logits = pl.pallas_call(
        _moe_body,
        grid=grid,
        in_specs=[
            pl.BlockSpec((TOK_TILE, hsz), lambda i: (i, 0)),
            pl.BlockSpec((hsz, LANES), lambda i: (0, 0)),
            pl.BlockSpec((hsz, LANES), lambda i: (0, 0)),
            pl.BlockSpec((1, LANES), lambda i: (0, 0)),
            pl.BlockSpec((NEXP, hsz, hsz), lambda i: (0, 0, 0)),
            pl.BlockSpec((NEXP, hsz), lambda i: (0, 0)),
        ],
        out_specs=[
            pl.BlockSpec((TOK_TILE, hsz), lambda i: (i, 0)),
            pl.BlockSpec((TOK_TILE, NEXP), lambda i: (i, 0)),
        ],
        out_shape=[
            jax.ShapeDtypeStruct((tokens, hsz), jnp.float32),
            jax.ShapeDtypeStruct((tokens, NEXP), jnp.float32),
        ],
    )(xf, gwt, gwt_lo, gb, w_bf, expert_b)
    return out.reshape(bsz, seq, hsz), logits
